# baseline (speedup 1.0000x reference)
import functools

import jax
import jax.numpy as jnp
from jax import lax
from jax.experimental import pallas as pl
from jax.experimental.pallas import tpu as pltpu
from jax.experimental.pallas import tpu_sc as plsc

BATCH = 16384
DIM = 32
CHUNK = 128
GROUP = 16


def _sc_gather_gmf(user, item, gmf_user4, gmf_item4, mlp_user4, mlp_item4):
    info = plsc.get_sparse_core_info()
    nc, ns = info.num_cores, info.num_subcores
    nw = nc * ns
    b_per_w = BATCH // nw
    n_chunks = b_per_w // CHUNK
    n_groups = CHUNK // GROUP
    rows_w = b_per_w // 4

    mesh = plsc.VectorSubcoreMesh(core_axis_name="c", subcore_axis_name="s")
    out_sds = jax.ShapeDtypeStruct((BATCH // 4, 128), jnp.float32)

    @functools.partial(
        pl.kernel,
        mesh=mesh,
        out_type=[out_sds, out_sds, out_sds],
        compiler_params=pltpu.CompilerParams(needs_layout_passes=False),
        scratch_types=[
            pltpu.VMEM((b_per_w,), jnp.int32),
            pltpu.VMEM((b_per_w,), jnp.int32),
            pltpu.VMEM((b_per_w,), jnp.int32),
            pltpu.VMEM((b_per_w,), jnp.int32),
            pltpu.VMEM((CHUNK, 128), jnp.float32),
            pltpu.VMEM((CHUNK, 128), jnp.float32),
            pltpu.VMEM((rows_w, 128), jnp.float32),
            pltpu.VMEM((rows_w, 128), jnp.float32),
            pltpu.VMEM((rows_w, 128), jnp.float32),
            pltpu.SemaphoreType.DMA,
        ],
    )
    def body(user_h, item_h, gu_h, gi_h, mu_h, mi_h,
             out_gmf, out_mu, out_mi,
             raw_u, raw_v, row_u, row_v, line_a, line_b,
             buf_gmf, buf_mu, buf_mi, sem):
        wid = lax.axis_index("s") * nc + lax.axis_index("c")
        base = wid * b_per_w

        pltpu.sync_copy(user_h.at[pl.ds(base, b_per_w)], raw_u)
        pltpu.sync_copy(item_h.at[pl.ds(base, b_per_w)], raw_v)

        def rows_body(k, carry):
            off = k * GROUP
            row_u[pl.ds(off, GROUP)] = lax.shift_right_logical(
                raw_u[pl.ds(off, GROUP)], 2)
            row_v[pl.ds(off, GROUP)] = lax.shift_right_logical(
                raw_v[pl.ds(off, GROUP)], 2)
            return carry

        lax.fori_loop(0, b_per_w // GROUP, rows_body, 0)

        lane = lax.iota(jnp.int32, GROUP)
        lane_row = lax.shift_right_logical(lane, 2)
        lane_col = (lane & 3) * DIM

        def extract_pair(c, emit):
            def group_body(g, carry):
                off = c * CHUNK + g * GROUP
                r_loc = g * GROUP + lane
                ca = (raw_u[pl.ds(off, GROUP)] & 3) * DIM
                cb = (raw_v[pl.ds(off, GROUP)] & 3) * DIM
                r_out = (off >> 2) + lane_row
                for j in range(DIM):
                    col = lane_col + j
                    a = plsc.load_gather(line_a, [r_loc, ca + j])
                    b = plsc.load_gather(line_b, [r_loc, cb + j])
                    emit(r_out, col, a, b)
                return carry
            lax.fori_loop(0, n_groups, group_body, 0)

        for c in range(n_chunks):
            idx_sl = pl.ds(c * CHUNK, CHUNK)
            pltpu.async_copy(gu_h.at[row_u.at[idx_sl]], line_a, sem).wait()
            pltpu.async_copy(gi_h.at[row_v.at[idx_sl]], line_b, sem).wait()

            def emit_gmf(r_out, col, a, b):
                plsc.store_scatter(buf_gmf, [r_out, col], a * b)

            extract_pair(c, emit_gmf)

        pltpu.sync_copy(buf_gmf, out_gmf.at[pl.ds(wid * rows_w, rows_w)])

        for c in range(n_chunks):
            idx_sl = pl.ds(c * CHUNK, CHUNK)
            pltpu.async_copy(mu_h.at[row_u.at[idx_sl]], line_a, sem).wait()
            pltpu.async_copy(mi_h.at[row_v.at[idx_sl]], line_b, sem).wait()

            def emit_mlp(r_out, col, a, b):
                plsc.store_scatter(buf_mu, [r_out, col], a)
                plsc.store_scatter(buf_mi, [r_out, col], b)

            extract_pair(c, emit_mlp)

        pltpu.sync_copy(buf_mu, out_mu.at[pl.ds(wid * rows_w, rows_w)])
        pltpu.sync_copy(buf_mi, out_mi.at[pl.ds(wid * rows_w, rows_w)])

    return body(user, item, gmf_user4, gmf_item4, mlp_user4, mlp_item4)


def _tc_mlp_body(gmf, xu, xi, w1a, w1b, b1, w2, b2, w3, b3, wg, wm, bfc, out):
    h = xu[:] @ w1a[:] + xi[:] @ w1b[:] + b1[:]
    h = jnp.maximum(h, 0.0)
    h = jnp.maximum(h @ w2[:] + b2[:], 0.0)
    h = jnp.maximum(h @ w3[:] + b3[:], 0.0)
    logit = (jnp.sum(gmf[:] * wg[:], axis=1, keepdims=True)
             + jnp.sum(h * wm[:], axis=1, keepdims=True) + bfc[:])
    out[:] = jax.nn.sigmoid(logit)


def _tc_mlp(gmf_prod, mlp_u, mlp_i, W1, b1, W2, b2, W3, b3, Wfc, bfc):
    blk = 2048
    grid = BATCH // blk
    data_spec = pl.BlockSpec((blk, DIM), lambda i: (i, 0))

    def whole(shape):
        return pl.BlockSpec(shape, lambda i: (0, 0))

    w1a = W1[:DIM]
    w1b = W1[DIM:]
    wg = Wfc[:DIM].reshape(1, DIM)
    wm = Wfc[DIM:].reshape(1, 8)

    out = pl.pallas_call(
        _tc_mlp_body,
        grid=(grid,),
        in_specs=[
            data_spec, data_spec, data_spec,
            whole((DIM, 32)), whole((DIM, 32)), whole((1, 32)),
            whole((32, 16)), whole((1, 16)),
            whole((16, 8)), whole((1, 8)),
            whole((1, DIM)), whole((1, 8)), whole((1, 1)),
        ],
        out_specs=pl.BlockSpec((blk, 1), lambda i: (i, 0)),
        out_shape=jax.ShapeDtypeStruct((BATCH, 1), jnp.float32),
    )(gmf_prod, mlp_u, mlp_i,
      w1a, w1b, b1.reshape(1, 32),
      W2, b2.reshape(1, 16),
      W3, b3.reshape(1, 8),
      wg, wm, bfc.reshape(1, 1))
    return out.reshape(BATCH)


def kernel(user, item, gmf_user, gmf_item, mlp_user, mlp_item,
           W1, b1, W2, b2, W3, b3, Wfc, bfc):
    user = user.astype(jnp.int32)
    item = item.astype(jnp.int32)
    gmf_p, mlp_u, mlp_i = _sc_gather_gmf(
        user, item,
        gmf_user.reshape(-1, 128), gmf_item.reshape(-1, 128),
        mlp_user.reshape(-1, 128), mlp_item.reshape(-1, 128))
    return _tc_mlp(gmf_p.reshape(BATCH, DIM), mlp_u.reshape(BATCH, DIM),
                   mlp_i.reshape(BATCH, DIM), W1, b1, W2, b2, W3, b3,
                   Wfc, bfc)


# speedup vs baseline: 5.4016x; 5.4016x over previous
import functools

import jax
import jax.numpy as jnp
from jax import lax
from jax.experimental import pallas as pl
from jax.experimental.pallas import tpu as pltpu
from jax.experimental.pallas import tpu_sc as plsc

BATCH = 16384
DIM = 32
NROWS = 1_000_000
CHUNK = 128
GROUP = 16
TBLK = 2048
TGRID = 123
NLINES = TGRID * TBLK
LANE_BLOCKS = -(-NROWS // TBLK)


def _tc_transpose_body(*refs):
    ins, outs = refs[:16], refs[16:]
    for side in range(2):
        pieces = []
        for a in range(4):
            for t in range(2):
                x = ins[4 * (2 * side + t) + a][:].astype(jnp.bfloat16)
                pieces.append(pltpu.bitcast(x, jnp.int32))
        outs[side][:] = jnp.concatenate(pieces, axis=0).T


def _tc_transpose(gmf_u, mlp_u, gmf_i, mlp_i):
    def mk_in_spec(a):
        return pl.BlockSpec(
            (DIM, TBLK), lambda i, _a=a: (0, jnp.minimum(i + TGRID * _a,
                                                         LANE_BLOCKS - 1)))

    in_specs = [mk_in_spec(a) for _ in range(4) for a in range(4)]
    out = pl.pallas_call(
        _tc_transpose_body,
        grid=(TGRID,),
        in_specs=in_specs,
        out_specs=[pl.BlockSpec((TBLK, 128), lambda i: (i, 0))] * 2,
        out_shape=[jax.ShapeDtypeStruct((NLINES, 128), jnp.int32)] * 2,
    )(*[t for t in (gmf_u, mlp_u, gmf_i, mlp_i) for _ in range(4)])
    return out


def _sc_gather_gmf(user, item, lines_u, lines_i):
    info = plsc.get_sparse_core_info()
    nc, ns = info.num_cores, info.num_subcores
    nw = nc * ns
    b_per_w = BATCH // nw
    n_chunks = b_per_w // CHUNK
    n_groups = CHUNK // GROUP
    rows_w = b_per_w // 4

    mesh = plsc.VectorSubcoreMesh(core_axis_name="c", subcore_axis_name="s")
    out_sds = jax.ShapeDtypeStruct((BATCH // 4, 128), jnp.float32)

    @functools.partial(
        pl.kernel,
        mesh=mesh,
        out_type=[out_sds, out_sds, out_sds],
        compiler_params=pltpu.CompilerParams(needs_layout_passes=False),
        scratch_types=[
            pltpu.VMEM((b_per_w,), jnp.int32),
            pltpu.VMEM((b_per_w,), jnp.int32),
            pltpu.VMEM((b_per_w,), jnp.int32),
            pltpu.VMEM((b_per_w,), jnp.int32),
            pltpu.VMEM((2, CHUNK, 128), jnp.int32),
            pltpu.VMEM((2, CHUNK, 128), jnp.int32),
            pltpu.VMEM((rows_w, 128), jnp.float32),
            pltpu.VMEM((rows_w, 128), jnp.float32),
            pltpu.VMEM((rows_w, 128), jnp.float32),
            pltpu.SemaphoreType.DMA,
        ],
    )
    def body(user_h, item_h, lu_h, li_h,
             out_gmf, out_mu, out_mi,
             row_u, row_v, col_u, col_v, line_a, line_b,
             buf_gmf, buf_mu, buf_mi, sem):
        wid = lax.axis_index("s") * nc + lax.axis_index("c")
        base = wid * b_per_w

        pltpu.sync_copy(user_h.at[pl.ds(base, b_per_w)], row_u)
        pltpu.sync_copy(item_h.at[pl.ds(base, b_per_w)], row_v)

        nl = jnp.full((GROUP,), NLINES, jnp.int32)

        def rows_body(k, carry):
            off = pl.ds(k * GROUP, GROUP)
            u = row_u[off]
            v = row_v[off]
            au = lax.div(u, nl)
            av = lax.div(v, nl)
            row_u[off] = u - au * NLINES
            row_v[off] = v - av * NLINES
            col_u[off] = au * DIM
            col_v[off] = av * DIM
            return carry

        lax.fori_loop(0, b_per_w // GROUP, rows_body, 0)

        lane = lax.iota(jnp.int32, GROUP)
        lane_row = lax.shift_right_logical(lane, 2)
        lane_col = (lane & 3) * DIM

        def fire(c, ph):
            sl = pl.ds(c * CHUNK, CHUNK)
            pltpu.async_copy(lu_h.at[row_u.at[sl]], line_a.at[ph], sem)
            pltpu.async_copy(li_h.at[row_v.at[sl]], line_b.at[ph], sem)

        def drain():
            for _ in range(2):
                pltpu.make_async_copy(
                    lu_h.at[pl.ds(0, CHUNK)], line_a.at[0], sem).wait()

        himask = jnp.full((GROUP,), -65536, jnp.int32)

        def lo(word):
            return plsc.bitcast(lax.shift_left(word, 16), jnp.float32)

        def hi(word):
            return plsc.bitcast(word & himask, jnp.float32)

        def extract(c, ph):
            phv = jnp.full((GROUP,), ph, jnp.int32)

            def group_body(g, carry):
                off = c * CHUNK + g * GROUP
                sl = pl.ds(off, GROUP)
                r_loc = g * GROUP + lane
                ca = col_u[sl]
                cb = col_v[sl]
                r_out = (off >> 2) + lane_row
                for j in range(16):
                    c_out = lane_col + 2 * j
                    gu = plsc.load_gather(line_a, [phv, r_loc, ca + j])
                    gi = plsc.load_gather(line_b, [phv, r_loc, cb + j])
                    plsc.store_scatter(buf_gmf, [r_out, c_out],
                                       lo(gu) * lo(gi))
                    plsc.store_scatter(buf_gmf, [r_out, c_out + 1],
                                       hi(gu) * hi(gi))
                    mu = plsc.load_gather(line_a, [phv, r_loc, ca + 16 + j])
                    mi = plsc.load_gather(line_b, [phv, r_loc, cb + 16 + j])
                    plsc.store_scatter(buf_mu, [r_out, c_out], lo(mu))
                    plsc.store_scatter(buf_mu, [r_out, c_out + 1], hi(mu))
                    plsc.store_scatter(buf_mi, [r_out, c_out], lo(mi))
                    plsc.store_scatter(buf_mi, [r_out, c_out + 1], hi(mi))
                return carry

            lax.fori_loop(0, n_groups, group_body, 0)

        fire(0, 0)
        for c in range(n_chunks):
            drain()
            if c + 1 < n_chunks:
                fire(c + 1, (c + 1) % 2)
            extract(c, c % 2)

        pltpu.sync_copy(buf_gmf, out_gmf.at[pl.ds(wid * rows_w, rows_w)])
        pltpu.sync_copy(buf_mu, out_mu.at[pl.ds(wid * rows_w, rows_w)])
        pltpu.sync_copy(buf_mi, out_mi.at[pl.ds(wid * rows_w, rows_w)])

    return body(user, item, lines_u, lines_i)


def _tc_mlp_body(gmf_p, xu_p, xi_p, w1a, w1b, b1, w2, b2, w3, b3,
                 wg, wm, bfc, out):
    f32 = jnp.float32
    h = xu_p[:] @ w1a[:] + xi_p[:] @ w1b[:] + b1[:]
    h = jnp.maximum(h, 0.0)
    h = jnp.maximum(jax.lax.dot(h, w2[:], preferred_element_type=f32) + b2[:],
                    0.0)
    h = jnp.maximum(jax.lax.dot(h, w3[:], preferred_element_type=f32) + b3[:],
                    0.0)
    logit = (jax.lax.dot(gmf_p[:], wg[:], preferred_element_type=f32)
             + jax.lax.dot(h, wm[:], preferred_element_type=f32) + bfc[:])
    out[:] = jax.nn.sigmoid(logit)


def _tc_mlp(gmf_p, mlp_u, mlp_i, W1, b1, W2, b2, W3, b3, Wfc, bfc):
    blk = 512
    grid = (BATCH // 4) // blk
    data_spec = pl.BlockSpec((blk, 128), lambda i: (i, 0))

    def whole(shape):
        return pl.BlockSpec(shape, lambda i: (0, 0))

    eye4 = jnp.eye(4, dtype=jnp.float32)
    w1a = jnp.kron(eye4, W1[:DIM])
    w1b = jnp.kron(eye4, W1[DIM:])
    w2 = jnp.kron(eye4, W2)
    w3 = jnp.kron(eye4, W3)
    wg = jnp.kron(eye4, Wfc[:DIM])
    wm = jnp.kron(eye4, Wfc[DIM:])
    b1t = jnp.tile(b1, 4).reshape(1, 128)
    b2t = jnp.tile(b2, 4).reshape(1, 64)
    b3t = jnp.tile(b3, 4).reshape(1, 32)

    out = pl.pallas_call(
        _tc_mlp_body,
        grid=(grid,),
        in_specs=[
            data_spec, data_spec, data_spec,
            whole((128, 128)), whole((128, 128)), whole((1, 128)),
            whole((128, 64)), whole((1, 64)),
            whole((64, 32)), whole((1, 32)),
            whole((128, 4)), whole((32, 4)), whole((1, 1)),
        ],
        out_specs=pl.BlockSpec((blk, 4), lambda i: (i, 0)),
        out_shape=jax.ShapeDtypeStruct((BATCH // 4, 4), jnp.float32),
    )(gmf_p, mlp_u, mlp_i,
      w1a, w1b, b1t, w2, b2t, w3, b3t, wg, wm, bfc.reshape(1, 1))
    return out.reshape(BATCH)


def kernel(user, item, gmf_user, gmf_item, mlp_user, mlp_item,
           W1, b1, W2, b2, W3, b3, Wfc, bfc):
    user = user.astype(jnp.int32)
    item = item.astype(jnp.int32)
    lines_u, lines_i = _tc_transpose(
        gmf_user.T, mlp_user.T, gmf_item.T, mlp_item.T)
    gmf_p, mlp_u, mlp_i = _sc_gather_gmf(user, item, lines_u, lines_i)
    return _tc_mlp(gmf_p, mlp_u, mlp_i, W1, b1, W2, b2, W3, b3, Wfc, bfc)


# speedup vs baseline: 5.6746x; 1.0505x over previous
import functools

import jax
import jax.numpy as jnp
from jax import lax
from jax.experimental import pallas as pl
from jax.experimental.pallas import tpu as pltpu
from jax.experimental.pallas import tpu_sc as plsc

BATCH = 16384
DIM = 32
NROWS = 1_000_000
CHUNK = 128
GROUP = 16
TBLK = 4096
TGRID = 62
NLINES = TGRID * TBLK
LANE_BLOCKS = -(-NROWS // TBLK)


def _tc_transpose_body(*refs):
    ins, outs = refs[:16], refs[16:]
    for side in range(2):
        pieces = []
        for a in range(4):
            for t in range(2):
                x = ins[4 * (2 * side + t) + a][:].astype(jnp.bfloat16)
                pieces.append(pltpu.bitcast(x, jnp.int32))
        outs[side][:] = jnp.concatenate(pieces, axis=0).T


def _tc_transpose(gmf_u, mlp_u, gmf_i, mlp_i):
    def mk_in_spec(a):
        return pl.BlockSpec(
            (DIM, TBLK), lambda i, _a=a: (0, jnp.minimum(i + TGRID * _a,
                                                         LANE_BLOCKS - 1)))

    in_specs = [mk_in_spec(a) for _ in range(4) for a in range(4)]
    out = pl.pallas_call(
        _tc_transpose_body,
        grid=(TGRID,),
        in_specs=in_specs,
        out_specs=[pl.BlockSpec((TBLK, 128), lambda i: (i, 0))] * 2,
        out_shape=[jax.ShapeDtypeStruct((NLINES, 128), jnp.int32)] * 2,
    )(*[t for t in (gmf_u, mlp_u, gmf_i, mlp_i) for _ in range(4)])
    return out


def _sc_gather_gmf(user, item, lines_u, lines_i):
    info = plsc.get_sparse_core_info()
    nc, ns = info.num_cores, info.num_subcores
    nw = nc * ns
    b_per_w = BATCH // nw
    n_chunks = b_per_w // CHUNK
    n_groups = CHUNK // GROUP
    rows_w = b_per_w // 4

    mesh = plsc.VectorSubcoreMesh(core_axis_name="c", subcore_axis_name="s")
    out_sds = jax.ShapeDtypeStruct((BATCH // 4, 128), jnp.float32)

    @functools.partial(
        pl.kernel,
        mesh=mesh,
        out_type=[out_sds, out_sds, out_sds],
        compiler_params=pltpu.CompilerParams(needs_layout_passes=False),
        scratch_types=[
            pltpu.VMEM((b_per_w,), jnp.int32),
            pltpu.VMEM((b_per_w,), jnp.int32),
            pltpu.VMEM((b_per_w,), jnp.int32),
            pltpu.VMEM((b_per_w,), jnp.int32),
            pltpu.VMEM((2, CHUNK, 128), jnp.int32),
            pltpu.VMEM((2, CHUNK, 128), jnp.int32),
            pltpu.VMEM((rows_w, 128), jnp.float32),
            pltpu.VMEM((rows_w, 128), jnp.float32),
            pltpu.VMEM((rows_w, 128), jnp.float32),
            pltpu.SemaphoreType.DMA,
        ],
    )
    def body(user_h, item_h, lu_h, li_h,
             out_gmf, out_mu, out_mi,
             row_u, row_v, col_u, col_v, line_a, line_b,
             buf_gmf, buf_mu, buf_mi, sem):
        wid = lax.axis_index("s") * nc + lax.axis_index("c")
        base = wid * b_per_w

        pltpu.sync_copy(user_h.at[pl.ds(base, b_per_w)], row_u)
        pltpu.sync_copy(item_h.at[pl.ds(base, b_per_w)], row_v)

        nl = jnp.full((GROUP,), NLINES, jnp.int32)

        def rows_body(k, carry):
            off = pl.ds(k * GROUP, GROUP)
            u = row_u[off]
            v = row_v[off]
            au = lax.div(u, nl)
            av = lax.div(v, nl)
            row_u[off] = u - au * NLINES
            row_v[off] = v - av * NLINES
            col_u[off] = au * DIM
            col_v[off] = av * DIM
            return carry

        lax.fori_loop(0, b_per_w // GROUP, rows_body, 0)

        lane = lax.iota(jnp.int32, GROUP)
        lane_row = lax.shift_right_logical(lane, 2)
        lane_col = (lane & 3) * DIM

        def fire(c, ph):
            sl = pl.ds(c * CHUNK, CHUNK)
            pltpu.async_copy(lu_h.at[row_u.at[sl]], line_a.at[ph], sem)
            pltpu.async_copy(li_h.at[row_v.at[sl]], line_b.at[ph], sem)

        def drain():
            for _ in range(2):
                pltpu.make_async_copy(
                    lu_h.at[pl.ds(0, CHUNK)], line_a.at[0], sem).wait()

        himask = jnp.full((GROUP,), -65536, jnp.int32)

        def lo(word):
            return plsc.bitcast(lax.shift_left(word, 16), jnp.float32)

        def hi(word):
            return plsc.bitcast(word & himask, jnp.float32)

        def extract(c, ph):
            phv = jnp.full((GROUP,), ph, jnp.int32)

            def group_body(g, carry):
                off = c * CHUNK + g * GROUP
                sl = pl.ds(off, GROUP)
                r_loc = g * GROUP + lane
                ca = col_u[sl]
                cb = col_v[sl]
                r_out = (off >> 2) + lane_row
                for j in range(16):
                    c_out = lane_col + 2 * j
                    gu = plsc.load_gather(line_a, [phv, r_loc, ca + j])
                    gi = plsc.load_gather(line_b, [phv, r_loc, cb + j])
                    plsc.store_scatter(buf_gmf, [r_out, c_out],
                                       lo(gu) * lo(gi))
                    plsc.store_scatter(buf_gmf, [r_out, c_out + 1],
                                       hi(gu) * hi(gi))
                    mu = plsc.load_gather(line_a, [phv, r_loc, ca + 16 + j])
                    mi = plsc.load_gather(line_b, [phv, r_loc, cb + 16 + j])
                    plsc.store_scatter(buf_mu, [r_out, c_out], lo(mu))
                    plsc.store_scatter(buf_mu, [r_out, c_out + 1], hi(mu))
                    plsc.store_scatter(buf_mi, [r_out, c_out], lo(mi))
                    plsc.store_scatter(buf_mi, [r_out, c_out + 1], hi(mi))
                return carry

            lax.fori_loop(0, n_groups, group_body, 0)

        fire(0, 0)
        for c in range(n_chunks):
            drain()
            if c + 1 < n_chunks:
                fire(c + 1, (c + 1) % 2)
            extract(c, c % 2)

        pltpu.sync_copy(buf_gmf, out_gmf.at[pl.ds(wid * rows_w, rows_w)])
        pltpu.sync_copy(buf_mu, out_mu.at[pl.ds(wid * rows_w, rows_w)])
        pltpu.sync_copy(buf_mi, out_mi.at[pl.ds(wid * rows_w, rows_w)])

    return body(user, item, lines_u, lines_i)


def _tc_mlp_body(gmf_p, xu_p, xi_p, w1a, w1b, b1, w2, b2, w3, b3,
                 wg, wm, bfc, out):
    f32 = jnp.float32
    h = xu_p[:] @ w1a[:] + xi_p[:] @ w1b[:] + b1[:]
    h = jnp.maximum(h, 0.0)
    h = jnp.maximum(jax.lax.dot(h, w2[:], preferred_element_type=f32) + b2[:],
                    0.0)
    h = jnp.maximum(jax.lax.dot(h, w3[:], preferred_element_type=f32) + b3[:],
                    0.0)
    logit = (jax.lax.dot(gmf_p[:], wg[:], preferred_element_type=f32)
             + jax.lax.dot(h, wm[:], preferred_element_type=f32) + bfc[:])
    out[:] = jax.nn.sigmoid(logit)


def _tc_mlp(gmf_p, mlp_u, mlp_i, W1, b1, W2, b2, W3, b3, Wfc, bfc):
    blk = 512
    grid = (BATCH // 4) // blk
    data_spec = pl.BlockSpec((blk, 128), lambda i: (i, 0))

    def whole(shape):
        return pl.BlockSpec(shape, lambda i: (0, 0))

    eye4 = jnp.eye(4, dtype=jnp.float32)
    w1a = jnp.kron(eye4, W1[:DIM])
    w1b = jnp.kron(eye4, W1[DIM:])
    w2 = jnp.kron(eye4, W2)
    w3 = jnp.kron(eye4, W3)
    wg = jnp.kron(eye4, Wfc[:DIM])
    wm = jnp.kron(eye4, Wfc[DIM:])
    b1t = jnp.tile(b1, 4).reshape(1, 128)
    b2t = jnp.tile(b2, 4).reshape(1, 64)
    b3t = jnp.tile(b3, 4).reshape(1, 32)

    out = pl.pallas_call(
        _tc_mlp_body,
        grid=(grid,),
        in_specs=[
            data_spec, data_spec, data_spec,
            whole((128, 128)), whole((128, 128)), whole((1, 128)),
            whole((128, 64)), whole((1, 64)),
            whole((64, 32)), whole((1, 32)),
            whole((128, 4)), whole((32, 4)), whole((1, 1)),
        ],
        out_specs=pl.BlockSpec((blk, 4), lambda i: (i, 0)),
        out_shape=jax.ShapeDtypeStruct((BATCH // 4, 4), jnp.float32),
    )(gmf_p, mlp_u, mlp_i,
      w1a, w1b, b1t, w2, b2t, w3, b3t, wg, wm, bfc.reshape(1, 1))
    return out.reshape(BATCH)


def kernel(user, item, gmf_user, gmf_item, mlp_user, mlp_item,
           W1, b1, W2, b2, W3, b3, Wfc, bfc):
    user = user.astype(jnp.int32)
    item = item.astype(jnp.int32)
    lines_u, lines_i = _tc_transpose(
        gmf_user.T, mlp_user.T, gmf_item.T, mlp_item.T)
    gmf_p, mlp_u, mlp_i = _sc_gather_gmf(user, item, lines_u, lines_i)
    return _tc_mlp(gmf_p, mlp_u, mlp_i, W1, b1, W2, b2, W3, b3, Wfc, bfc)
